# baseline (speedup 1.0000x reference)
import functools

import jax
import jax.numpy as jnp
from jax import lax
from jax.experimental import pallas as pl
from jax.experimental.pallas import tpu as pltpu
from jax.experimental.pallas import tpu_sc as plsc

NC = 2
NS = 16
NW = NC * NS
BLK = 128
STG = 8
NODE_BLK = 512


def _sc_mesh():
  return plsc.VectorSubcoreMesh(core_axis_name="c", subcore_axis_name="s")


def _make_deg_kernel(npad, nb, nst, rpt):
  @functools.partial(
      pl.kernel,
      out_type=jax.ShapeDtypeStruct((2 * npad,), jnp.float32),
      mesh=_sc_mesh(),
      scratch_types=[
          pltpu.VMEM((4 * STG, BLK), jnp.int32),
          pltpu.VMEM((BLK,), jnp.float32),
          pltpu.VMEM((rpt,), jnp.float32),
          pltpu.VMEM_SHARED((npad,), jnp.float32),
          pltpu.SemaphoreType.DMA,
          pltpu.SemaphoreType.DMA,
      ],
  )
  def deg_kernel(dst3_h, ones_h, z_h, out_h, dst_st, ones_v, buf_v, acc,
                 ssem, isem):
    c = lax.axis_index("c")
    s = lax.axis_index("s")
    w = c * NS + s
    nq = 4 * STG
    nquad = nb // nq
    pltpu.sync_copy(ones_h, ones_v)
    pltpu.sync_copy(z_h, buf_v)
    pltpu.sync_copy(buf_v, acc.at[pl.ds(s * rpt, rpt)])
    plsc.subcore_barrier()

    def quad(i, carry):
      i1 = pltpu.async_copy(dst3_h.at[pl.ds(w * nb + i * nq, nq)],
                            dst_st, isem)
      i1.wait()
      sds = [pltpu.async_copy(ones_v, acc.at[dst_st.at[j]], ssem, add=True)
             for j in range(nq)]
      for d in sds:
        d.wait()
      return carry

    lax.fori_loop(0, nquad, quad, 0)
    plsc.subcore_barrier()
    pltpu.sync_copy(acc.at[pl.ds(s * rpt, rpt)], buf_v)
    pltpu.sync_copy(buf_v, out_h.at[pl.ds(c * npad + s * rpt, rpt)])

  return deg_kernel


def _make_sraw_kernel(npad, nb, nst, rpt):
  @functools.partial(
      pl.kernel,
      out_type=jax.ShapeDtypeStruct((2 * npad,), jnp.float32),
      mesh=_sc_mesh(),
      scratch_types=[
          pltpu.VMEM((4 * STG, BLK), jnp.int32),
          pltpu.VMEM((4 * STG, BLK), jnp.int32),
          [pltpu.VMEM((BLK,), jnp.float32)] * (4 * STG),
          pltpu.VMEM((rpt,), jnp.float32),
          pltpu.VMEM_SHARED((npad,), jnp.float32),
          pltpu.SemaphoreType.DMA,
          pltpu.SemaphoreType.DMA,
          pltpu.SemaphoreType.DMA,
      ],
  )
  def sraw_kernel(src3_h, dst3_h, xs_h, z_h, out_h,
                  src_st, dst_st, rows, buf_v, acc, gsem, ssem, isem):
    c = lax.axis_index("c")
    s = lax.axis_index("s")
    w = c * NS + s
    nq = 4 * STG
    nquad = nb // nq
    pltpu.sync_copy(z_h, buf_v)
    pltpu.sync_copy(buf_v, acc.at[pl.ds(s * rpt, rpt)])
    plsc.subcore_barrier()

    def quad(i, carry):
      base = w * nb + i * nq
      i1 = pltpu.async_copy(src3_h.at[pl.ds(base, nq)], src_st, isem)
      i2 = pltpu.async_copy(dst3_h.at[pl.ds(base, nq)], dst_st, isem)
      i1.wait()
      i2.wait()
      g1 = [pltpu.async_copy(xs_h.at[src_st.at[j]], rows[j], gsem)
            for j in range(nq // 2)]
      for d in g1:
        d.wait()
      s1 = [pltpu.async_copy(rows[j], acc.at[dst_st.at[j]], ssem, add=True)
            for j in range(nq // 2)]
      g2 = [pltpu.async_copy(xs_h.at[src_st.at[j]], rows[j], gsem)
            for j in range(nq // 2, nq)]
      for d in g2:
        d.wait()
      s2 = [pltpu.async_copy(rows[j], acc.at[dst_st.at[j]], ssem, add=True)
            for j in range(nq // 2, nq)]
      for d in s1 + s2:
        d.wait()
      return carry

    lax.fori_loop(0, nquad, quad, 0)
    plsc.subcore_barrier()
    pltpu.sync_copy(acc.at[pl.ds(s * rpt, rpt)], buf_v)
    pltpu.sync_copy(buf_v, out_h.at[pl.ds(c * npad + s * rpt, rpt)])

  return sraw_kernel


def _make_g1s_kernel(npad, rpt, h, nch):
  fc = h // nch
  npb = 224

  @functools.partial(
      pl.kernel,
      out_type=jax.ShapeDtypeStruct((nch * npad, fc), jnp.float32),
      mesh=_sc_mesh(),
      scratch_types=[
          pltpu.VMEM((npb,), jnp.float32),
          pltpu.VMEM((npb,), jnp.float32),
          pltpu.VMEM((h,), jnp.float32),
          pltpu.VMEM((h,), jnp.float32),
          [pltpu.VMEM((npb, fc), jnp.float32)] * nch,
      ],
      compiler_params=pltpu.CompilerParams(use_tc_tiling_on_sc=False),
  )
  def g1s_kernel(s_h, dinv_h, w1_h, b1_h, table_h,
                 s_v, dinv_v, w1_v, b1_v, tbufs):
    c = lax.axis_index("c")
    s = lax.axis_index("s")
    w = c * NS + s
    pltpu.sync_copy(w1_h, w1_v)
    pltpu.sync_copy(b1_h, b1_v)
    w1s = [w1_v[pl.ds(k * 16, 16)] for k in range(h // 16)]
    b1s = [b1_v[pl.ds(k * 16, 16)] for k in range(h // 16)]
    per_w = npad // NW
    nbatch = per_w // npb
    assert per_w % npb == 0

    def batch(b, carry):
      base = w * per_w + b * npb
      pltpu.sync_copy(s_h.at[pl.ds(base, npb)], s_v)
      pltpu.sync_copy(dinv_h.at[pl.ds(base, npb)], dinv_v)

      def group(g, carry2):
        sv16 = s_v[pl.ds(g * 16, 16)]
        dv16 = dinv_v[pl.ds(g * 16, 16)]
        for l in range(16):
          sv = sv16[l]
          dv = dv16[l]
          for p, tb in enumerate(tbufs):
            for k in range(fc // 16):
              f = p * (fc // 16) + k
              v = jnp.maximum(sv * w1s[f] + b1s[f], 0.0) * dv
              tb[g * 16 + l, pl.ds(k * 16, 16)] = v
        return carry2

      lax.fori_loop(0, npb // 16, group, 0)
      for p, tb in enumerate(tbufs):
        pltpu.sync_copy(tb, table_h.at[pl.ds(p * npad + base, npb)])
      return carry

    lax.fori_loop(0, nbatch, batch, 0)

  return g1s_kernel


def _make_agg_kernel(npad, nb, nst, rpt, h, nch):
  fc = h // nch
  GB = STG // 4

  @functools.partial(
      pl.kernel,
      out_type=jax.ShapeDtypeStruct((2 * npad, h), jnp.float32),
      mesh=_sc_mesh(),
      scratch_types=[
          pltpu.VMEM((2 * GB * BLK,), jnp.int32),
          pltpu.VMEM((2, GB, BLK), jnp.int32),
          [pltpu.VMEM((GB * BLK, fc), jnp.float32)] * 2,
          pltpu.VMEM((rpt // 32, fc), jnp.float32),
          pltpu.VMEM((rpt // 32, fc), jnp.float32),
          pltpu.VMEM_SHARED((npad, fc), jnp.float32),
          pltpu.SemaphoreType.DMA,
          pltpu.SemaphoreType.DMA,
          pltpu.SemaphoreType.DMA,
          pltpu.SemaphoreType.DMA,
      ],
      compiler_params=pltpu.CompilerParams(use_tc_tiling_on_sc=False),
  )
  def agg_kernel(src1_h, dst3_h, table_h, z_h, out_h,
                 src_st, dst_st, rows, zbuf, dbuf, acc,
                 gsem, ssem, gsem2, isem):
    c = lax.axis_index("c")
    s = lax.axis_index("s")
    w = c * NS + s
    sub = rpt // 32
    npair = nb // (2 * GB)

    def chunk(p, carry0):
      off = p * npad
      for q in range(32):
        pltpu.sync_copy(zbuf, acc.at[pl.ds(s * rpt + q * sub, sub)])
      plsc.subcore_barrier()

      def drain(sem, b):
        pltpu.make_async_copy(table_h.at[pl.ds(0, GB * BLK)],
                              rows[b], sem).wait()

      def half(i, b, sem):
        base = w * nb + (i * 2 + b) * GB
        i1 = pltpu.async_copy(src1_h.at[pl.ds(base * BLK, GB * BLK)],
                              src_st.at[pl.ds(b * GB * BLK, GB * BLK)],
                              isem)
        i2 = pltpu.async_copy(dst3_h.at[pl.ds(base, GB)], dst_st.at[b],
                              isem)
        i1.wait()
        i2.wait()
        tview = table_h.at[pl.ds(off, npad)]
        pltpu.async_copy(
            tview.at[src_st.at[pl.ds(b * GB * BLK, GB * BLK)]],
            rows[b], gsem).wait()
        for j in range(GB):
          pltpu.async_copy(rows[b].at[pl.ds(j * BLK, BLK)],
                           acc.at[dst_st.at[b].at[j]], sem, add=True)

      def pair(i, carry):
        @pl.when(i > 0)
        def _():
          drain(ssem, 0)
        half(i, 0, ssem)

        @pl.when(i > 0)
        def _():
          drain(gsem2, 1)
        half(i, 1, gsem2)
        return carry

      lax.fori_loop(0, npair, pair, 0)
      drain(ssem, 0)
      drain(gsem2, 1)
      plsc.subcore_barrier()
      for q in range(32):
        pltpu.sync_copy(acc.at[pl.ds(s * rpt + q * sub, sub)], dbuf)
        pltpu.sync_copy(
            dbuf,
            out_h.at[pl.ds(c * npad + s * rpt + q * sub, sub),
                     pl.ds(p * fc, fc)])
      return carry0

    pltpu.sync_copy(z_h, zbuf)
    lax.fori_loop(0, nch, chunk, 0)

  return agg_kernel


def _tc_final_body(parts_ref, dinv_ref, w2_ref, b2_ref, wf_ref, bf_ref, o_ref):
  ps = parts_ref[...]
  dv = dinv_ref[...]
  agg = (ps[0] + ps[1]) * dv
  h2 = jnp.maximum(
      jnp.dot(agg, w2_ref[...], preferred_element_type=jnp.float32)
      + b2_ref[...], 0.0)
  o_ref[...] = (jnp.dot(h2, wf_ref[...], preferred_element_type=jnp.float32)
                + bf_ref[...])


def kernel(x, edge_index, W1, b1, W2, b2, Wf, bf):
  n = x.shape[0]
  e = edge_index.shape[1]
  h = W1.shape[1]
  out_w = Wf.shape[1]
  nch = 4
  fc = h // nch

  npad = ((n + 1 + NODE_BLK - 1) // NODE_BLK) * NODE_BLK
  rpt = npad // NS
  ea = e + n
  egrain = NW * BLK * STG * 4
  epad = ((ea + egrain - 1) // egrain) * egrain
  nb = epad // (NW * BLK)
  nst = nb // STG
  grid_n = npad // NODE_BLK

  loops = jnp.arange(n, dtype=jnp.int32)
  pad_idx = n + jnp.arange(epad - ea, dtype=jnp.int32) % (npad - n)
  src1 = jnp.concatenate([edge_index[0], loops, pad_idx])
  src3 = src1.reshape(-1, BLK)
  dst3 = jnp.concatenate([edge_index[1], loops, pad_idx]).reshape(-1, BLK)
  x_pad = jnp.concatenate(
      [x[:, 0], jnp.zeros((npad - n,), jnp.float32)])

  ones128 = jnp.ones((BLK,), jnp.float32)
  z1 = jnp.zeros((rpt,), jnp.float32)
  z32 = jnp.zeros((rpt // 32, fc), jnp.float32)

  deg2 = _make_deg_kernel(npad, nb, nst, rpt)(dst3, ones128, z1)
  deg = deg2[:npad] + deg2[npad:]
  dinv = jnp.where(deg > 0, lax.rsqrt(jnp.maximum(deg, 1e-12)), 0.0)
  xs = dinv * x_pad

  sr2 = _make_sraw_kernel(npad, nb, nst, rpt)(src3, dst3, xs, z1)
  s = dinv * (sr2[:npad] + sr2[npad:])

  g1s = _make_g1s_kernel(npad, rpt, h, nch)(
      s, dinv, W1.reshape(h), b1)

  parts = _make_agg_kernel(npad, nb, nst, rpt, h, nch)(
      src1, dst3, g1s, z32)

  out = pl.pallas_call(
      _tc_final_body,
      grid=(grid_n,),
      in_specs=[
          pl.BlockSpec((2, NODE_BLK, h), lambda i: (0, i, 0)),
          pl.BlockSpec((NODE_BLK, 1), lambda i: (i, 0)),
          pl.BlockSpec((h, h), lambda i: (0, 0)),
          pl.BlockSpec((1, h), lambda i: (0, 0)),
          pl.BlockSpec((h, out_w), lambda i: (0, 0)),
          pl.BlockSpec((1, out_w), lambda i: (0, 0)),
      ],
      out_specs=pl.BlockSpec((NODE_BLK, out_w), lambda i: (i, 0)),
      out_shape=jax.ShapeDtypeStruct((npad, out_w), jnp.float32),
  )(parts.reshape(2, npad, h), dinv.reshape(npad, 1),
    W2, b2.reshape(1, h), Wf, bf.reshape(1, out_w))

  return out[:n]


# speedup vs baseline: 1.1098x; 1.1098x over previous
import functools

import jax
import jax.numpy as jnp
from jax import lax
from jax.experimental import pallas as pl
from jax.experimental.pallas import tpu as pltpu
from jax.experimental.pallas import tpu_sc as plsc

NC = 2
NS = 16
NW = NC * NS
BLK = 128
STG = 8
NODE_BLK = 512


def _sc_mesh():
  return plsc.VectorSubcoreMesh(core_axis_name="c", subcore_axis_name="s")


def _make_deg_kernel(npad, nb, nst, rpt):
  @functools.partial(
      pl.kernel,
      out_type=jax.ShapeDtypeStruct((2 * npad,), jnp.float32),
      mesh=_sc_mesh(),
      scratch_types=[
          pltpu.VMEM((4 * STG, BLK), jnp.int32),
          pltpu.VMEM((BLK,), jnp.float32),
          pltpu.VMEM((rpt,), jnp.float32),
          pltpu.VMEM_SHARED((npad,), jnp.float32),
          pltpu.SemaphoreType.DMA,
          pltpu.SemaphoreType.DMA,
      ],
  )
  def deg_kernel(dst3_h, ones_h, z_h, out_h, dst_st, ones_v, buf_v, acc,
                 ssem, isem):
    c = lax.axis_index("c")
    s = lax.axis_index("s")
    w = c * NS + s
    nq = 4 * STG
    nquad = nb // nq
    pltpu.sync_copy(ones_h, ones_v)
    pltpu.sync_copy(z_h, buf_v)
    pltpu.sync_copy(buf_v, acc.at[pl.ds(s * rpt, rpt)])
    plsc.subcore_barrier()

    def quad(i, carry):
      i1 = pltpu.async_copy(dst3_h.at[pl.ds(w * nb + i * nq, nq)],
                            dst_st, isem)
      i1.wait()
      sds = [pltpu.async_copy(ones_v, acc.at[dst_st.at[j]], ssem, add=True)
             for j in range(nq)]
      for d in sds:
        d.wait()
      return carry

    lax.fori_loop(0, nquad, quad, 0)
    plsc.subcore_barrier()
    pltpu.sync_copy(acc.at[pl.ds(s * rpt, rpt)], buf_v)
    pltpu.sync_copy(buf_v, out_h.at[pl.ds(c * npad + s * rpt, rpt)])

  return deg_kernel


def _make_sraw_kernel(npad, nb, nst, rpt):
  @functools.partial(
      pl.kernel,
      out_type=jax.ShapeDtypeStruct((2 * npad,), jnp.float32),
      mesh=_sc_mesh(),
      scratch_types=[
          pltpu.VMEM((4 * STG, BLK), jnp.int32),
          pltpu.VMEM((4 * STG, BLK), jnp.int32),
          [pltpu.VMEM((BLK,), jnp.float32)] * (4 * STG),
          pltpu.VMEM((rpt,), jnp.float32),
          pltpu.VMEM_SHARED((npad,), jnp.float32),
          pltpu.SemaphoreType.DMA,
          pltpu.SemaphoreType.DMA,
          pltpu.SemaphoreType.DMA,
      ],
  )
  def sraw_kernel(src3_h, dst3_h, xs_h, z_h, out_h,
                  src_st, dst_st, rows, buf_v, acc, gsem, ssem, isem):
    c = lax.axis_index("c")
    s = lax.axis_index("s")
    w = c * NS + s
    nq = 4 * STG
    nquad = nb // nq
    pltpu.sync_copy(z_h, buf_v)
    pltpu.sync_copy(buf_v, acc.at[pl.ds(s * rpt, rpt)])
    plsc.subcore_barrier()

    def quad(i, carry):
      base = w * nb + i * nq
      i1 = pltpu.async_copy(src3_h.at[pl.ds(base, nq)], src_st, isem)
      i2 = pltpu.async_copy(dst3_h.at[pl.ds(base, nq)], dst_st, isem)
      i1.wait()
      i2.wait()
      g1 = [pltpu.async_copy(xs_h.at[src_st.at[j]], rows[j], gsem)
            for j in range(nq // 2)]
      for d in g1:
        d.wait()
      s1 = [pltpu.async_copy(rows[j], acc.at[dst_st.at[j]], ssem, add=True)
            for j in range(nq // 2)]
      g2 = [pltpu.async_copy(xs_h.at[src_st.at[j]], rows[j], gsem)
            for j in range(nq // 2, nq)]
      for d in g2:
        d.wait()
      s2 = [pltpu.async_copy(rows[j], acc.at[dst_st.at[j]], ssem, add=True)
            for j in range(nq // 2, nq)]
      for d in s1 + s2:
        d.wait()
      return carry

    lax.fori_loop(0, nquad, quad, 0)
    plsc.subcore_barrier()
    pltpu.sync_copy(acc.at[pl.ds(s * rpt, rpt)], buf_v)
    pltpu.sync_copy(buf_v, out_h.at[pl.ds(c * npad + s * rpt, rpt)])

  return sraw_kernel


def _make_g1s_kernel(npad, rpt, h, nch):
  fc = h // nch
  npb = 224

  @functools.partial(
      pl.kernel,
      out_type=jax.ShapeDtypeStruct((nch * npad, fc), jnp.float32),
      mesh=_sc_mesh(),
      scratch_types=[
          pltpu.VMEM((npb,), jnp.float32),
          pltpu.VMEM((npb,), jnp.float32),
          pltpu.VMEM((h,), jnp.float32),
          pltpu.VMEM((h,), jnp.float32),
          [pltpu.VMEM((npb, fc), jnp.float32)] * nch,
      ],
      compiler_params=pltpu.CompilerParams(use_tc_tiling_on_sc=False),
  )
  def g1s_kernel(s_h, dinv_h, w1_h, b1_h, table_h,
                 s_v, dinv_v, w1_v, b1_v, tbufs):
    c = lax.axis_index("c")
    s = lax.axis_index("s")
    w = c * NS + s
    pltpu.sync_copy(w1_h, w1_v)
    pltpu.sync_copy(b1_h, b1_v)
    w1s = [w1_v[pl.ds(k * 16, 16)] for k in range(h // 16)]
    b1s = [b1_v[pl.ds(k * 16, 16)] for k in range(h // 16)]
    per_w = npad // NW
    nbatch = per_w // npb
    assert per_w % npb == 0

    def batch(b, carry):
      base = w * per_w + b * npb
      pltpu.sync_copy(s_h.at[pl.ds(base, npb)], s_v)
      pltpu.sync_copy(dinv_h.at[pl.ds(base, npb)], dinv_v)

      def group(g, carry2):
        sv16 = s_v[pl.ds(g * 16, 16)]
        dv16 = dinv_v[pl.ds(g * 16, 16)]
        for l in range(16):
          sv = sv16[l]
          dv = dv16[l]
          for p, tb in enumerate(tbufs):
            for k in range(fc // 16):
              f = p * (fc // 16) + k
              v = jnp.maximum(sv * w1s[f] + b1s[f], 0.0) * dv
              tb[g * 16 + l, pl.ds(k * 16, 16)] = v
        return carry2

      lax.fori_loop(0, npb // 16, group, 0)
      for p, tb in enumerate(tbufs):
        pltpu.sync_copy(tb, table_h.at[pl.ds(p * npad + base, npb)])
      return carry

    lax.fori_loop(0, nbatch, batch, 0)

  return g1s_kernel


def _make_agg_kernel(npad, nb, nst, rpt, h, nch):
  fc = h // nch
  GB = STG

  @functools.partial(
      pl.kernel,
      out_type=jax.ShapeDtypeStruct((2 * npad, h), jnp.float32),
      mesh=_sc_mesh(),
      scratch_types=[
          pltpu.VMEM((2 * GB * BLK,), jnp.int32),
          pltpu.VMEM((2, GB, BLK), jnp.int32),
          [pltpu.VMEM((GB * BLK, fc), jnp.float32)] * 2,
          pltpu.VMEM((rpt // 8, fc), jnp.float32),
          pltpu.VMEM((rpt // 8, fc), jnp.float32),
          pltpu.VMEM_SHARED((npad, fc), jnp.float32),
          pltpu.SemaphoreType.DMA,
          pltpu.SemaphoreType.DMA,
          pltpu.SemaphoreType.DMA,
          pltpu.SemaphoreType.DMA,
      ],
      compiler_params=pltpu.CompilerParams(use_tc_tiling_on_sc=False),
  )
  def agg_kernel(src1_h, dst3_h, table_h, z_h, out_h,
                 src_st, dst_st, rows, zbuf, dbuf, acc,
                 gsem, ssem, gsem2, isem):
    c = lax.axis_index("c")
    s = lax.axis_index("s")
    w = c * NS + s
    sub = rpt // 8
    npair = nb // (2 * GB)

    def chunk(p, carry0):
      off = p * npad
      for q in range(8):
        pltpu.sync_copy(zbuf, acc.at[pl.ds(s * rpt + q * sub, sub)])
      plsc.subcore_barrier()

      def drain(sem, b):
        pltpu.make_async_copy(table_h.at[pl.ds(0, GB * BLK)],
                              rows[b], sem).wait()

      def half(i, b, sem):
        base = w * nb + (i * 2 + b) * GB
        i1 = pltpu.async_copy(src1_h.at[pl.ds(base * BLK, GB * BLK)],
                              src_st.at[pl.ds(b * GB * BLK, GB * BLK)],
                              isem)
        i2 = pltpu.async_copy(dst3_h.at[pl.ds(base, GB)], dst_st.at[b],
                              isem)
        i1.wait()
        i2.wait()
        tview = table_h.at[pl.ds(off, npad)]
        pltpu.async_copy(
            tview.at[src_st.at[pl.ds(b * GB * BLK, GB * BLK)]],
            rows[b], gsem).wait()
        for j in range(GB):
          pltpu.async_copy(rows[b].at[pl.ds(j * BLK, BLK)],
                           acc.at[dst_st.at[b].at[j]], sem, add=True)

      def pair(i, carry):
        @pl.when(i > 0)
        def _():
          drain(ssem, 0)
        half(i, 0, ssem)

        @pl.when(i > 0)
        def _():
          drain(gsem2, 1)
        half(i, 1, gsem2)
        return carry

      lax.fori_loop(0, npair, pair, 0)
      drain(ssem, 0)
      drain(gsem2, 1)
      plsc.subcore_barrier()
      for q in range(8):
        pltpu.sync_copy(acc.at[pl.ds(s * rpt + q * sub, sub)], dbuf)
        pltpu.sync_copy(
            dbuf,
            out_h.at[pl.ds(c * npad + s * rpt + q * sub, sub),
                     pl.ds(p * fc, fc)])
      return carry0

    pltpu.sync_copy(z_h, zbuf)
    lax.fori_loop(0, nch, chunk, 0)

  return agg_kernel


def _tc_final_body(parts_ref, dinv_ref, w2_ref, b2_ref, wf_ref, bf_ref, o_ref):
  ps = parts_ref[...]
  dv = dinv_ref[...]
  agg = (ps[0] + ps[1]) * dv
  h2 = jnp.maximum(
      jnp.dot(agg, w2_ref[...], preferred_element_type=jnp.float32)
      + b2_ref[...], 0.0)
  o_ref[...] = (jnp.dot(h2, wf_ref[...], preferred_element_type=jnp.float32)
                + bf_ref[...])


def kernel(x, edge_index, W1, b1, W2, b2, Wf, bf):
  n = x.shape[0]
  e = edge_index.shape[1]
  h = W1.shape[1]
  out_w = Wf.shape[1]
  nch = 8
  fc = h // nch

  npad = ((n + 1 + NODE_BLK - 1) // NODE_BLK) * NODE_BLK
  rpt = npad // NS
  ea = e + n
  egrain = NW * BLK * STG * 4
  epad = ((ea + egrain - 1) // egrain) * egrain
  nb = epad // (NW * BLK)
  nst = nb // STG
  grid_n = npad // NODE_BLK

  loops = jnp.arange(n, dtype=jnp.int32)
  pad_idx = n + jnp.arange(epad - ea, dtype=jnp.int32) % (npad - n)
  src1 = jnp.concatenate([edge_index[0], loops, pad_idx])
  src3 = src1.reshape(-1, BLK)
  dst3 = jnp.concatenate([edge_index[1], loops, pad_idx]).reshape(-1, BLK)
  x_pad = jnp.concatenate(
      [x[:, 0], jnp.zeros((npad - n,), jnp.float32)])

  ones128 = jnp.ones((BLK,), jnp.float32)
  z1 = jnp.zeros((rpt,), jnp.float32)
  z32 = jnp.zeros((rpt // 8, fc), jnp.float32)

  deg2 = _make_deg_kernel(npad, nb, nst, rpt)(dst3, ones128, z1)
  deg = deg2[:npad] + deg2[npad:]
  dinv = jnp.where(deg > 0, lax.rsqrt(jnp.maximum(deg, 1e-12)), 0.0)
  xs = dinv * x_pad

  sr2 = _make_sraw_kernel(npad, nb, nst, rpt)(src3, dst3, xs, z1)
  s = dinv * (sr2[:npad] + sr2[npad:])

  g1s = _make_g1s_kernel(npad, rpt, h, nch)(
      s, dinv, W1.reshape(h), b1)

  parts = _make_agg_kernel(npad, nb, nst, rpt, h, nch)(
      src1, dst3, g1s, z32)

  out = pl.pallas_call(
      _tc_final_body,
      grid=(grid_n,),
      in_specs=[
          pl.BlockSpec((2, NODE_BLK, h), lambda i: (0, i, 0)),
          pl.BlockSpec((NODE_BLK, 1), lambda i: (i, 0)),
          pl.BlockSpec((h, h), lambda i: (0, 0)),
          pl.BlockSpec((1, h), lambda i: (0, 0)),
          pl.BlockSpec((h, out_w), lambda i: (0, 0)),
          pl.BlockSpec((1, out_w), lambda i: (0, 0)),
      ],
      out_specs=pl.BlockSpec((NODE_BLK, out_w), lambda i: (i, 0)),
      out_shape=jax.ShapeDtypeStruct((npad, out_w), jnp.float32),
  )(parts.reshape(2, npad, h), dinv.reshape(npad, 1),
    W2, b2.reshape(1, h), Wf, bf.reshape(1, out_w))

  return out[:n]
